# baseline (speedup 1.0000x reference)
import functools

import jax
import jax.numpy as jnp
from jax import lax
from jax.experimental import pallas as pl
from jax.experimental.pallas import tpu as pltpu
from jax.experimental.pallas import tpu_sc as plsc

_B = 4096
_C = 1000
_CP = 1024
_BB = 512
_GRID = _B // _BB


def _argmax_body(x_ref, coarse_ref, fine_ref, idx_ref, m_ref, reg_ref):
    i = pl.program_id(0)
    x = x_ref[...]
    idx_ref[0, 0, :] = jnp.argmax(x, axis=1).astype(jnp.int32)

    @pl.when(i == 0)
    def _():
        fine = fine_ref[...]
        m_ref[:, : _C] = coarse_ref[...][:, None] * fine
        m_ref[:, _C:] = jnp.ones((_C, _CP - _C), jnp.float32)
        reg_ref[0, 0] = jnp.sum(jnp.abs(fine - 1.0)) / (_C * _C)


def _phase_a(x, coarse, fine):
    idx3, m_ext, reg = pl.pallas_call(
        _argmax_body,
        grid=(_GRID,),
        in_specs=[
            pl.BlockSpec((_BB, _C), lambda i: (i, 0)),
            pl.BlockSpec((_C,), lambda i: (0,)),
            pl.BlockSpec((_C, _C), lambda i: (0, 0)),
        ],
        out_specs=[
            pl.BlockSpec((1, 1, _BB), lambda i: (i, 0, 0)),
            pl.BlockSpec((_C, _CP), lambda i: (0, 0)),
            pl.BlockSpec(memory_space=pltpu.SMEM),
        ],
        out_shape=[
            jax.ShapeDtypeStruct((_GRID, 1, _BB), jnp.int32),
            jax.ShapeDtypeStruct((_C, _CP), jnp.float32),
            jax.ShapeDtypeStruct((1, 1), jnp.float32),
        ],
    )(x, coarse, fine)
    return idx3.reshape(_B), m_ext, reg


_NC = 2
_NS = 16
_NW = _NC * _NS
_BPW = _B // _NW
_SUB = 2
_BPS = _BPW // _SUB


def _sc_gather(m_ext, idx):
    mesh = plsc.VectorSubcoreMesh(core_axis_name="c", subcore_axis_name="s")

    @functools.partial(
        pl.kernel,
        mesh=mesh,
        out_type=jax.ShapeDtypeStruct((_B, _CP), jnp.float32),
        scratch_types=[
            pltpu.VMEM((_BPW,), jnp.int32),
            pltpu.VMEM((_BPS, _CP), jnp.float32),
            pltpu.SemaphoreType.DMA,
        ],
    )
    def k(m_hbm, idx_hbm, out_hbm, idx_v, rows_v, sem):
        wid = lax.axis_index("s") * _NC + lax.axis_index("c")
        base = wid * _BPW
        pltpu.sync_copy(idx_hbm.at[pl.ds(base, _BPW)], idx_v)
        for s in range(_SUB):
            pltpu.async_copy(
                m_hbm.at[idx_v.at[pl.ds(s * _BPS, _BPS)]], rows_v, sem
            ).wait()
            pltpu.sync_copy(
                rows_v, out_hbm.at[pl.ds(base + s * _BPS, _BPS)]
            )

    return k(m_ext, idx)


def _consume_body(x_ref, r_ref, lab_ref, reg_ref, sv_ref, loss_ref):
    i = pl.program_id(0)
    x = x_ref[...]
    denom = r_ref[...][:, : _C]
    sv = x / denom
    sv_ref[...] = sv

    lab = lab_ref[0, 0, :].astype(jnp.int32)
    classes = jax.lax.broadcasted_iota(jnp.int32, (_BB, _C), 1)
    sel = jnp.sum(jnp.where(lab[:, None] == classes, sv, 0.0), axis=1)
    row_max = jnp.max(sv, axis=1)
    lse = row_max + jnp.log(jnp.sum(jnp.exp(sv - row_max[:, None]), axis=1))
    part = jnp.sum(lse - sel)

    @pl.when(i == 0)
    def _():
        loss_ref[0, 0] = reg_ref[0, 0]

    loss_ref[0, 0] += part / _B


def _phase_c(x, r, labels3, reg):
    sv, loss = pl.pallas_call(
        _consume_body,
        grid=(_GRID,),
        in_specs=[
            pl.BlockSpec((_BB, _C), lambda i: (i, 0)),
            pl.BlockSpec((_BB, _CP), lambda i: (i, 0)),
            pl.BlockSpec((1, 1, _BB), lambda i: (i, 0, 0)),
            pl.BlockSpec(memory_space=pltpu.SMEM),
        ],
        out_specs=[
            pl.BlockSpec((_BB, _C), lambda i: (i, 0)),
            pl.BlockSpec(memory_space=pltpu.SMEM),
        ],
        out_shape=[
            jax.ShapeDtypeStruct((_B, _C), jnp.float32),
            jax.ShapeDtypeStruct((1, 1), jnp.float32),
        ],
    )(x, r, labels3, reg)
    return sv, loss


def kernel(Simple_vector, label_list, coarse_scaling_vector, fine_scaling_matrix):
    idx, m_ext, reg = _phase_a(
        Simple_vector, coarse_scaling_vector, fine_scaling_matrix
    )
    r = _sc_gather(m_ext, idx)
    labels3 = label_list.reshape(_GRID, 1, _BB)
    sv, loss = _phase_c(Simple_vector, r, labels3, reg)
    softmaxed = jnp.zeros((), dtype=sv.dtype)
    return (sv, loss.reshape(()), softmaxed)


# speedup vs baseline: 1.6337x; 1.6337x over previous
import functools

import jax
import jax.numpy as jnp
from jax.experimental import pallas as pl
from jax.experimental.pallas import tpu as pltpu

_B = 4096
_C = 1000
_BB = 1024
_GRID = _B // _BB


def _fused_body(x_ref, lab_ref, coarse_ref, fine_ref, sv_ref, loss_ref, m_ref):
    i = pl.program_id(0)

    @pl.when(i == 0)
    def _():
        m_ref[...] = coarse_ref[...][:, None] * fine_ref[...]

    x = x_ref[...]
    idx = jnp.argmax(x, axis=1).astype(jnp.int32)
    classes = jax.lax.broadcasted_iota(jnp.int32, (_BB, _C), 1)
    onehot = (idx[:, None] == classes).astype(jnp.float32)
    denom = jnp.dot(onehot, m_ref[...], preferred_element_type=jnp.float32)
    sv = x / denom
    sv_ref[...] = sv

    lab = lab_ref[0, 0, :].astype(jnp.int32)
    lab_onehot = lab[:, None] == classes
    sel = jnp.sum(jnp.where(lab_onehot, sv, 0.0), axis=1)
    row_max = jnp.max(sv, axis=1)
    lse = row_max + jnp.log(jnp.sum(jnp.exp(sv - row_max[:, None]), axis=1))
    part = jnp.sum(lse - sel)

    @pl.when(i == 0)
    def _():
        reg = jnp.sum(jnp.abs(fine_ref[...] - 1.0))
        loss_ref[0, 0] = reg / (_C * _C)

    loss_ref[0, 0] += part / _B


def kernel(Simple_vector, label_list, coarse_scaling_vector, fine_scaling_matrix):
    labels3 = label_list.reshape(_GRID, 1, _BB)
    sv, loss = pl.pallas_call(
        _fused_body,
        grid=(_GRID,),
        in_specs=[
            pl.BlockSpec((_BB, _C), lambda i: (i, 0)),
            pl.BlockSpec((1, 1, _BB), lambda i: (i, 0, 0)),
            pl.BlockSpec((_C,), lambda i: (0,)),
            pl.BlockSpec((_C, _C), lambda i: (0, 0)),
        ],
        out_specs=[
            pl.BlockSpec((_BB, _C), lambda i: (i, 0)),
            pl.BlockSpec(memory_space=pltpu.SMEM),
        ],
        out_shape=[
            jax.ShapeDtypeStruct((_B, _C), jnp.float32),
            jax.ShapeDtypeStruct((1, 1), jnp.float32),
        ],
        scratch_shapes=[pltpu.VMEM((_C, _C), jnp.float32)],
    )(Simple_vector, labels3, coarse_scaling_vector, fine_scaling_matrix)
    softmaxed = jnp.zeros((), dtype=sv.dtype)
    return (sv, loss.reshape(()), softmaxed)


# speedup vs baseline: 3.4051x; 2.0842x over previous
import jax
import jax.numpy as jnp
from jax.experimental import pallas as pl
from jax.experimental.pallas import tpu as pltpu

_B = 4096
_C = 1000
_BB = 512
_GRID = _B // _BB


def _fused_body(xt_ref, lab_ref, coarse_ref, fine_ref, svt_ref, loss_ref):
    i = pl.program_id(0)
    xt = xt_ref[...]
    idx = jnp.argmax(xt, axis=0).astype(jnp.int32)
    classes = jax.lax.broadcasted_iota(jnp.int32, (_C, _BB), 0)
    onehot = jnp.where(classes == idx[None, :], coarse_ref[...], 0.0)
    denom = jax.lax.dot_general(
        fine_ref[...], onehot, (((0,), (0,)), ((), ())),
        preferred_element_type=jnp.float32,
    )
    svt = xt / denom
    svt_ref[...] = svt

    lab = lab_ref[0, 0, :].astype(jnp.int32)
    sel = jnp.sum(jnp.where(lab[None, :] == classes, svt, 0.0), axis=0)
    col_max = jnp.max(svt, axis=0)
    lse = col_max + jnp.log(jnp.sum(jnp.exp(svt - col_max[None, :]), axis=0))
    part = jnp.sum(lse - sel)

    @pl.when(i == 0)
    def _():
        reg = jnp.sum(jnp.abs(fine_ref[...] - 1.0))
        loss_ref[0, 0] = reg / (_C * _C)

    loss_ref[0, 0] += part / _B


def kernel(Simple_vector, label_list, coarse_scaling_vector, fine_scaling_matrix):
    labels3 = label_list.reshape(_GRID, 1, _BB)
    svt, loss = pl.pallas_call(
        _fused_body,
        grid=(_GRID,),
        in_specs=[
            pl.BlockSpec((_C, _BB), lambda i: (0, i)),
            pl.BlockSpec((1, 1, _BB), lambda i: (i, 0, 0)),
            pl.BlockSpec((_C, 1), lambda i: (0, 0)),
            pl.BlockSpec((_C, _C), lambda i: (0, 0)),
        ],
        out_specs=[
            pl.BlockSpec((_C, _BB), lambda i: (0, i)),
            pl.BlockSpec(memory_space=pltpu.SMEM),
        ],
        out_shape=[
            jax.ShapeDtypeStruct((_C, _B), jnp.float32),
            jax.ShapeDtypeStruct((1, 1), jnp.float32),
        ],
    )(
        Simple_vector.T,
        labels3,
        coarse_scaling_vector[:, None],
        fine_scaling_matrix,
    )
    softmaxed = jnp.zeros((), dtype=svt.dtype)
    return (svt.T, loss.reshape(()), softmaxed)


# speedup vs baseline: 3.4322x; 1.0080x over previous
import jax
import jax.numpy as jnp
from jax.experimental import pallas as pl
from jax.experimental.pallas import tpu as pltpu

_B = 4096
_C = 1000
_BB = 512
_GRID = _B // _BB


def _fused_body(xt_ref, lab_ref, coarse_ref, fine_ref, svt_ref, loss_ref, fb_ref):
    i = pl.program_id(0)

    @pl.when(i == 0)
    def _():
        fb_ref[...] = fine_ref[...].astype(jnp.bfloat16)

    xt = xt_ref[...]
    idx = jnp.argmax(xt, axis=0).astype(jnp.int32)
    classes = jax.lax.broadcasted_iota(jnp.int32, (_C, _BB), 0)
    onehot = jnp.where(
        classes == idx[None, :], coarse_ref[...], 0.0
    ).astype(jnp.bfloat16)
    denom = jax.lax.dot_general(
        fb_ref[...], onehot, (((0,), (0,)), ((), ())),
        preferred_element_type=jnp.float32,
    )
    svt = xt / denom
    svt_ref[...] = svt

    lab = lab_ref[0, 0, :].astype(jnp.int32)
    sel = jnp.sum(jnp.where(lab[None, :] == classes, svt, 0.0), axis=0)
    col_max = jnp.max(svt, axis=0)
    lse = col_max + jnp.log(jnp.sum(jnp.exp(svt - col_max[None, :]), axis=0))
    part = jnp.sum(lse - sel)

    @pl.when(i == 0)
    def _():
        reg = jnp.sum(jnp.abs(fine_ref[...] - 1.0))
        loss_ref[0, 0] = reg / (_C * _C)

    loss_ref[0, 0] += part / _B


def kernel(Simple_vector, label_list, coarse_scaling_vector, fine_scaling_matrix):
    labels3 = label_list.reshape(_GRID, 1, _BB)
    svt, loss = pl.pallas_call(
        _fused_body,
        grid=(_GRID,),
        in_specs=[
            pl.BlockSpec((_C, _BB), lambda i: (0, i)),
            pl.BlockSpec((1, 1, _BB), lambda i: (i, 0, 0)),
            pl.BlockSpec((_C, 1), lambda i: (0, 0)),
            pl.BlockSpec((_C, _C), lambda i: (0, 0)),
        ],
        out_specs=[
            pl.BlockSpec((_C, _BB), lambda i: (0, i)),
            pl.BlockSpec(memory_space=pltpu.SMEM),
        ],
        out_shape=[
            jax.ShapeDtypeStruct((_C, _B), jnp.float32),
            jax.ShapeDtypeStruct((1, 1), jnp.float32),
        ],
        scratch_shapes=[pltpu.VMEM((_C, _C), jnp.bfloat16)],
    )(
        Simple_vector.T,
        labels3,
        coarse_scaling_vector[:, None],
        fine_scaling_matrix,
    )
    softmaxed = jnp.zeros((), dtype=svt.dtype)
    return (svt.T, loss.reshape(()), softmaxed)


# speedup vs baseline: 3.8308x; 1.1161x over previous
import jax
import jax.numpy as jnp
from jax.experimental import pallas as pl
from jax.experimental.pallas import tpu as pltpu

_B = 4096
_C = 1000
_BB = 1024
_GRID = _B // _BB


def _fused_body(xt_ref, lab_ref, coarse_ref, fine_ref, svt_ref, loss_ref, fb_ref):
    i = pl.program_id(0)

    @pl.when(i == 0)
    def _():
        fb_ref[...] = fine_ref[...].astype(jnp.bfloat16)

    xt = xt_ref[...]
    idx = jnp.argmax(xt, axis=0).astype(jnp.int32)
    classes = jax.lax.broadcasted_iota(jnp.int32, (_C, _BB), 0)
    onehot = jnp.where(
        classes == idx[None, :], coarse_ref[...], 0.0
    ).astype(jnp.bfloat16)
    denom = jax.lax.dot_general(
        fb_ref[...], onehot, (((0,), (0,)), ((), ())),
        preferred_element_type=jnp.float32,
    )
    svt = xt / denom
    svt_ref[...] = svt

    lab = lab_ref[0, 0, :].astype(jnp.int32)
    sel = jnp.sum(jnp.where(lab[None, :] == classes, svt, 0.0), axis=0)
    col_max = jnp.max(svt, axis=0)
    lse = col_max + jnp.log(jnp.sum(jnp.exp(svt - col_max[None, :]), axis=0))
    part = jnp.sum(lse - sel)

    @pl.when(i == 0)
    def _():
        reg = jnp.sum(jnp.abs(fine_ref[...] - 1.0))
        loss_ref[0, 0] = reg / (_C * _C)

    loss_ref[0, 0] += part / _B


def kernel(Simple_vector, label_list, coarse_scaling_vector, fine_scaling_matrix):
    labels3 = label_list.reshape(_GRID, 1, _BB)
    svt, loss = pl.pallas_call(
        _fused_body,
        grid=(_GRID,),
        in_specs=[
            pl.BlockSpec((_C, _BB), lambda i: (0, i)),
            pl.BlockSpec((1, 1, _BB), lambda i: (i, 0, 0)),
            pl.BlockSpec((_C, 1), lambda i: (0, 0)),
            pl.BlockSpec((_C, _C), lambda i: (0, 0)),
        ],
        out_specs=[
            pl.BlockSpec((_C, _BB), lambda i: (0, i)),
            pl.BlockSpec(memory_space=pltpu.SMEM),
        ],
        out_shape=[
            jax.ShapeDtypeStruct((_C, _B), jnp.float32),
            jax.ShapeDtypeStruct((1, 1), jnp.float32),
        ],
        scratch_shapes=[pltpu.VMEM((_C, _C), jnp.bfloat16)],
    )(
        Simple_vector.T,
        labels3,
        coarse_scaling_vector[:, None],
        fine_scaling_matrix,
    )
    softmaxed = jnp.zeros((), dtype=svt.dtype)
    return (svt.T, loss.reshape(()), softmaxed)
